# Initial kernel scaffold; baseline (speedup 1.0000x reference)
import jax
import jax.numpy as jnp
from jax.experimental import pallas as pl


def kernel(x, edge_index, W1, b1, W2, b2, Wp, bp):
    raise NotImplementedError("write your pallas kernel here")


# speedup vs baseline: 12.0466x; 12.0466x over previous
import functools

import jax
import jax.numpy as jnp
from jax import lax
from jax.experimental import pallas as pl
from jax.experimental.pallas import tpu as pltpu
from jax.experimental.pallas import tpu_sc as plsc

N_NODES = 10000
N_EDGES = 320000
D = 128
DD = 128

NC = 2
NS = 16
NW = NC * NS
EPW = N_EDGES // NW
K = 80
NCHUNK = EPW // K
RPT = 624
CH = 16

_mesh = plsc.VectorSubcoreMesh(core_axis_name="c", subcore_axis_name="s")


def _fill_rows(ref, nrows, width, value):
    v = jnp.full((16,), value, jnp.float32)
    groups = width // 16

    def body(i, _):
        r = i // groups
        g = i % groups
        ref[r, pl.ds(g * 16, 16)] = v
        return 0

    lax.fori_loop(0, nrows * groups, body, 0)


def _zero_rows(ref, nrows, width):
    _fill_rows(ref, nrows, width, 0.0)


def _tile_rows(s):
    base = s * RPT
    ntrips = jnp.where(s == NS - 1, (N_NODES - (NS - 1) * RPT) // CH,
                       RPT // CH)
    return base, ntrips


def _zero_acc(s, zb, acc):
    base, ntrips = _tile_rows(s)

    def z(i, _):
        pltpu.sync_copy(zb, acc.at[pl.ds(base + i * CH, CH)])
        return 0

    lax.fori_loop(0, ntrips, z, 0)


def _writeout_acc(c, s, zb, acc, out_hbm):
    base, ntrips = _tile_rows(s)

    def z(i, _):
        off = base + i * CH
        pltpu.sync_copy(acc.at[pl.ds(off, CH)], zb)
        pltpu.sync_copy(zb, out_hbm.at[c, pl.ds(off, CH)])
        return 0

    lax.fori_loop(0, ntrips, z, 0)


def _sc_degree_body(dst_hbm, out_hbm, idx_d, ones_v, zb, accd, sem):
    c = lax.axis_index("c")
    s = lax.axis_index("s")
    wid = s * NC + c

    _fill_rows(ones_v, K, DD, 1.0)
    _zero_rows(zb, CH, DD)

    _zero_acc(s, zb, accd)
    plsc.subcore_barrier()

    base = wid * EPW

    def chunk(i, _):
        pltpu.sync_copy(dst_hbm.at[pl.ds(base + i * K, K)], idx_d)
        pltpu.sync_copy(ones_v, accd.at[idx_d], add=True)
        return 0

    lax.fori_loop(0, NCHUNK, chunk, 0)
    plsc.subcore_barrier()

    _writeout_acc(c, s, zb, accd, out_hbm)


_sc_degree = pl.kernel(
    _sc_degree_body,
    out_type=jax.ShapeDtypeStruct((NC, N_NODES, DD), jnp.float32),
    mesh=_mesh,
    scratch_types=[
        pltpu.VMEM((K,), jnp.int32),
        pltpu.VMEM((K, DD), jnp.float32),
        pltpu.VMEM((CH, DD), jnp.float32),
        pltpu.VMEM_SHARED((N_NODES, DD), jnp.float32),
        pltpu.SemaphoreType.DMA,
    ],
)


def _sc_aggregate_body(t_hbm, src_hbm, dst_hbm, out_hbm,
                       idx_s, idx_d, rows, zb, acc, sem):
    c = lax.axis_index("c")
    s = lax.axis_index("s")
    wid = s * NC + c

    _zero_rows(zb, CH, D)
    _zero_acc(s, zb, acc)
    plsc.subcore_barrier()

    base = wid * EPW

    def chunk(i, _):
        e = base + i * K
        pltpu.sync_copy(src_hbm.at[pl.ds(e, K)], idx_s)
        pltpu.sync_copy(dst_hbm.at[pl.ds(e, K)], idx_d)
        pltpu.async_copy(t_hbm.at[idx_s], rows, sem).wait()
        pltpu.sync_copy(rows, acc.at[idx_d], add=True)
        return 0

    lax.fori_loop(0, NCHUNK, chunk, 0)
    plsc.subcore_barrier()

    _writeout_acc(c, s, zb, acc, out_hbm)


_sc_aggregate = pl.kernel(
    _sc_aggregate_body,
    out_type=jax.ShapeDtypeStruct((NC, N_NODES, D), jnp.float32),
    mesh=_mesh,
    scratch_types=[
        pltpu.VMEM((K,), jnp.int32),
        pltpu.VMEM((K,), jnp.int32),
        pltpu.VMEM((K, D), jnp.float32),
        pltpu.VMEM((CH, D), jnp.float32),
        pltpu.VMEM_SHARED((N_NODES, D), jnp.float32),
        pltpu.SemaphoreType.DMA,
    ],
)


BR = 1000
GRID = N_NODES // BR


def _dis_block(degp_ref):
    d = 1.0 + degp_ref[0, :, 0:1] + degp_ref[1, :, 0:1]
    return lax.rsqrt(d)


def _t1_body(x_ref, w_ref, degp_ref, o_ref):
    dis = _dis_block(degp_ref)
    o_ref[...] = dis * jnp.dot(x_ref[...], w_ref[...],
                               preferred_element_type=jnp.float32)


def _t2_body(p_ref, t_ref, degp_ref, b_ref, w_ref, o_ref):
    dis = _dis_block(degp_ref)
    h = p_ref[0] + p_ref[1] + t_ref[...]
    h = jnp.maximum(dis * h + b_ref[...], 0.0)
    o_ref[...] = dis * jnp.dot(h, w_ref[...],
                               preferred_element_type=jnp.float32)


def _t3_body(p_ref, t_ref, degp_ref, b_ref, w_ref, bp_ref, o_ref):
    dis = _dis_block(degp_ref)
    h = p_ref[0] + p_ref[1] + t_ref[...]
    h = jnp.maximum(dis * h + b_ref[...], 0.0)
    o_ref[...] = jnp.dot(h, w_ref[...],
                         preferred_element_type=jnp.float32) + bp_ref[...]


def _spec_rows(width):
    return pl.BlockSpec((BR, width), lambda i: (i, 0))


def _spec_pair(width):
    return pl.BlockSpec((NC, BR, width), lambda i: (0, i, 0))


def _spec_full(shape):
    return pl.BlockSpec(shape, lambda i: tuple(0 for _ in shape))


_t1 = pl.pallas_call(
    _t1_body,
    grid=(GRID,),
    in_specs=[_spec_rows(D), _spec_full((D, D)), _spec_pair(DD)],
    out_specs=_spec_rows(D),
    out_shape=jax.ShapeDtypeStruct((N_NODES, D), jnp.float32),
)

_t2 = pl.pallas_call(
    _t2_body,
    grid=(GRID,),
    in_specs=[_spec_pair(D), _spec_rows(D), _spec_pair(DD),
              _spec_full((1, D)), _spec_full((D, D))],
    out_specs=_spec_rows(D),
    out_shape=jax.ShapeDtypeStruct((N_NODES, D), jnp.float32),
)

_t3 = pl.pallas_call(
    _t3_body,
    grid=(GRID,),
    in_specs=[_spec_pair(D), _spec_rows(D), _spec_pair(DD),
              _spec_full((1, D)), _spec_full((D, 1)), _spec_full((1, 1))],
    out_specs=_spec_rows(1),
    out_shape=jax.ShapeDtypeStruct((N_NODES, 1), jnp.float32),
)


@jax.jit
def kernel(x, edge_index, W1, b1, W2, b2, Wp, bp):
    src = edge_index[0].astype(jnp.int32)
    dst = edge_index[1].astype(jnp.int32)
    b1r = b1.reshape(1, D)
    b2r = b2.reshape(1, D)
    bpr = bp.reshape(1, 1)

    degp = _sc_degree(dst)
    t1 = _t1(x, W1, degp)
    p1 = _sc_aggregate(t1, src, dst)
    t2 = _t2(p1, t1, degp, b1r, W2)
    p2 = _sc_aggregate(t2, src, dst)
    out = _t3(p2, t2, degp, b2r, Wp, bpr)
    return out


# speedup vs baseline: 19.0024x; 1.5774x over previous
import functools

import jax
import jax.numpy as jnp
from jax import lax
from jax.experimental import pallas as pl
from jax.experimental.pallas import tpu as pltpu
from jax.experimental.pallas import tpu_sc as plsc

N_NODES = 10000
N_EDGES = 320000
D = 128
DD = 128

NC = 2
NS = 16
NW = NC * NS
EPW = N_EDGES // NW
K = 80
NCHUNK = EPW // K
RPT = 624
CH = 16

_mesh = plsc.VectorSubcoreMesh(core_axis_name="c", subcore_axis_name="s")


def _fill_rows(ref, nrows, width, value):
    v = jnp.full((16,), value, jnp.float32)
    groups = width // 16

    def body(i, _):
        r = i // groups
        g = i % groups
        ref[r, pl.ds(g * 16, 16)] = v
        return 0

    lax.fori_loop(0, nrows * groups, body, 0)


def _zero_rows(ref, nrows, width):
    _fill_rows(ref, nrows, width, 0.0)


def _tile_rows(s):
    base = s * RPT
    ntrips = jnp.where(s == NS - 1, (N_NODES - (NS - 1) * RPT) // CH,
                       RPT // CH)
    return base, ntrips


def _zero_acc(s, zb, acc):
    base, ntrips = _tile_rows(s)

    def z(i, _):
        pltpu.sync_copy(zb, acc.at[pl.ds(base + i * CH, CH)])
        return 0

    lax.fori_loop(0, ntrips, z, 0)


def _writeout_acc(c, s, zb, acc, out_hbm):
    base, ntrips = _tile_rows(s)

    def z(i, _):
        off = base + i * CH
        pltpu.sync_copy(acc.at[pl.ds(off, CH)], zb)
        pltpu.sync_copy(zb, out_hbm.at[c, pl.ds(off, CH)])
        return 0

    lax.fori_loop(0, ntrips, z, 0)


def _sc_degree_body(dst_hbm, out_hbm, idx_d0, idx_d1, ones_v, zb, accd,
                    sem0, sem1):
    c = lax.axis_index("c")
    s = lax.axis_index("s")
    wid = s * NC + c

    _fill_rows(ones_v, K, DD, 1.0)
    _zero_rows(zb, CH, DD)

    _zero_acc(s, zb, accd)
    plsc.subcore_barrier()

    base = wid * EPW

    def start(j, idx_d, sem):
        pltpu.async_copy(dst_hbm.at[pl.ds(base + j * K, K)], idx_d, sem)

    def finish(j, idx_d, sem):
        pltpu.make_async_copy(dst_hbm.at[pl.ds(base + j * K, K)],
                              idx_d, sem).wait()
        pltpu.sync_copy(ones_v, accd.at[idx_d], add=True)

    start(0, idx_d0, sem0)

    def pair(i, _):
        start(2 * i + 1, idx_d1, sem1)
        finish(2 * i, idx_d0, sem0)
        start(2 * i + 2, idx_d0, sem0)
        finish(2 * i + 1, idx_d1, sem1)
        return 0

    lax.fori_loop(0, (NCHUNK - 1) // 2, pair, 0)
    finish(NCHUNK - 1, idx_d0, sem0)
    plsc.subcore_barrier()

    _writeout_acc(c, s, zb, accd, out_hbm)


_sc_degree = pl.kernel(
    _sc_degree_body,
    out_type=jax.ShapeDtypeStruct((NC, N_NODES, DD), jnp.float32),
    mesh=_mesh,
    scratch_types=[
        pltpu.VMEM((K,), jnp.int32),
        pltpu.VMEM((K,), jnp.int32),
        pltpu.VMEM((K, DD), jnp.float32),
        pltpu.VMEM((CH, DD), jnp.float32),
        pltpu.VMEM_SHARED((N_NODES, DD), jnp.float32),
        pltpu.SemaphoreType.DMA,
        pltpu.SemaphoreType.DMA,
    ],
)


def _sc_aggregate_body(t_hbm, src_hbm, dst_hbm, out_hbm,
                       idx_s0, idx_d0, idx_s1, idx_d1,
                       rows0, rows1, zb, acc, sem0, sem1):
    c = lax.axis_index("c")
    s = lax.axis_index("s")
    wid = s * NC + c

    _zero_rows(zb, CH, D)
    _zero_acc(s, zb, acc)
    plsc.subcore_barrier()

    base = wid * EPW

    def start(j, idx_s, idx_d, rows, sem):
        e = base + j * K
        pltpu.sync_copy(src_hbm.at[pl.ds(e, K)], idx_s)
        pltpu.sync_copy(dst_hbm.at[pl.ds(e, K)], idx_d)
        pltpu.async_copy(t_hbm.at[idx_s], rows, sem)

    def finish(idx_s, idx_d, rows, sem):
        pltpu.make_async_copy(t_hbm.at[idx_s], rows, sem).wait()
        pltpu.sync_copy(rows, acc.at[idx_d], add=True)

    start(0, idx_s0, idx_d0, rows0, sem0)

    def pair(i, _):
        start(2 * i + 1, idx_s1, idx_d1, rows1, sem1)
        finish(idx_s0, idx_d0, rows0, sem0)
        start(2 * i + 2, idx_s0, idx_d0, rows0, sem0)
        finish(idx_s1, idx_d1, rows1, sem1)
        return 0

    lax.fori_loop(0, (NCHUNK - 1) // 2, pair, 0)
    finish(idx_s0, idx_d0, rows0, sem0)
    plsc.subcore_barrier()

    _writeout_acc(c, s, zb, acc, out_hbm)


_sc_aggregate = pl.kernel(
    _sc_aggregate_body,
    out_type=jax.ShapeDtypeStruct((NC, N_NODES, D), jnp.float32),
    mesh=_mesh,
    scratch_types=[
        pltpu.VMEM((K,), jnp.int32),
        pltpu.VMEM((K,), jnp.int32),
        pltpu.VMEM((K,), jnp.int32),
        pltpu.VMEM((K,), jnp.int32),
        pltpu.VMEM((K, D), jnp.float32),
        pltpu.VMEM((K, D), jnp.float32),
        pltpu.VMEM((CH, D), jnp.float32),
        pltpu.VMEM_SHARED((N_NODES, D), jnp.float32),
        pltpu.SemaphoreType.DMA,
        pltpu.SemaphoreType.DMA,
    ],
)


BR = 1000
GRID = N_NODES // BR


def _dis_block(degp_ref):
    d = 1.0 + degp_ref[0, :, 0:1] + degp_ref[1, :, 0:1]
    return lax.rsqrt(d)


def _t1_body(x_ref, w_ref, degp_ref, o_ref):
    dis = _dis_block(degp_ref)
    o_ref[...] = dis * jnp.dot(x_ref[...], w_ref[...],
                               preferred_element_type=jnp.float32)


def _t2_body(p_ref, t_ref, degp_ref, b_ref, w_ref, o_ref):
    dis = _dis_block(degp_ref)
    h = p_ref[0] + p_ref[1] + t_ref[...]
    h = jnp.maximum(dis * h + b_ref[...], 0.0)
    o_ref[...] = dis * jnp.dot(h, w_ref[...],
                               preferred_element_type=jnp.float32)


def _t3_body(p_ref, t_ref, degp_ref, b_ref, w_ref, bp_ref, o_ref):
    dis = _dis_block(degp_ref)
    h = p_ref[0] + p_ref[1] + t_ref[...]
    h = jnp.maximum(dis * h + b_ref[...], 0.0)
    o_ref[...] = jnp.dot(h, w_ref[...],
                         preferred_element_type=jnp.float32) + bp_ref[...]


def _spec_rows(width):
    return pl.BlockSpec((BR, width), lambda i: (i, 0))


def _spec_pair(width):
    return pl.BlockSpec((NC, BR, width), lambda i: (0, i, 0))


def _spec_full(shape):
    return pl.BlockSpec(shape, lambda i: tuple(0 for _ in shape))


_t1 = pl.pallas_call(
    _t1_body,
    grid=(GRID,),
    in_specs=[_spec_rows(D), _spec_full((D, D)), _spec_pair(DD)],
    out_specs=_spec_rows(D),
    out_shape=jax.ShapeDtypeStruct((N_NODES, D), jnp.float32),
)

_t2 = pl.pallas_call(
    _t2_body,
    grid=(GRID,),
    in_specs=[_spec_pair(D), _spec_rows(D), _spec_pair(DD),
              _spec_full((1, D)), _spec_full((D, D))],
    out_specs=_spec_rows(D),
    out_shape=jax.ShapeDtypeStruct((N_NODES, D), jnp.float32),
)

_t3 = pl.pallas_call(
    _t3_body,
    grid=(GRID,),
    in_specs=[_spec_pair(D), _spec_rows(D), _spec_pair(DD),
              _spec_full((1, D)), _spec_full((D, 1)), _spec_full((1, 1))],
    out_specs=_spec_rows(1),
    out_shape=jax.ShapeDtypeStruct((N_NODES, 1), jnp.float32),
)


@jax.jit
def kernel(x, edge_index, W1, b1, W2, b2, Wp, bp):
    src = edge_index[0].astype(jnp.int32)
    dst = edge_index[1].astype(jnp.int32)
    b1r = b1.reshape(1, D)
    b2r = b2.reshape(1, D)
    bpr = bp.reshape(1, 1)

    degp = _sc_degree(dst)
    t1 = _t1(x, W1, degp)
    p1 = _sc_aggregate(t1, src, dst)
    t2 = _t2(p1, t1, degp, b1r, W2)
    p2 = _sc_aggregate(t2, src, dst)
    out = _t3(p2, t2, degp, b2r, Wp, bpr)
    return out


# speedup vs baseline: 25.1769x; 1.3249x over previous
import functools

import jax
import jax.numpy as jnp
from jax import lax
from jax.experimental import pallas as pl
from jax.experimental.pallas import tpu as pltpu
from jax.experimental.pallas import tpu_sc as plsc

N_NODES = 10000
N_EDGES = 320000
D = 128
DD = 128

NC = 2
NS = 16
NW = NC * NS
EPW = N_EDGES // NW
K = 80
NCHUNK = EPW // K
RPT = 624
CH = 16

_mesh = plsc.VectorSubcoreMesh(core_axis_name="c", subcore_axis_name="s")


def _fill_rows(ref, nrows, width, value):
    v = jnp.full((16,), value, jnp.float32)
    groups = width // 16

    def body(i, _):
        r = i // groups
        g = i % groups
        ref[r, pl.ds(g * 16, 16)] = v
        return 0

    lax.fori_loop(0, nrows * groups, body, 0)


def _zero_rows(ref, nrows, width):
    _fill_rows(ref, nrows, width, 0.0)


def _tile_rows(s):
    base = s * RPT
    ntrips = jnp.where(s == NS - 1, (N_NODES - (NS - 1) * RPT) // CH,
                       RPT // CH)
    return base, ntrips


def _zero_acc(s, zb, acc):
    base, ntrips = _tile_rows(s)

    def z(i, _):
        pltpu.sync_copy(zb, acc.at[pl.ds(base + i * CH, CH)])
        return 0

    lax.fori_loop(0, ntrips, z, 0)


def _writeout_acc(c, s, zb, acc, out_hbm):
    base, ntrips = _tile_rows(s)

    def z(i, _):
        off = base + i * CH
        pltpu.sync_copy(acc.at[pl.ds(off, CH)], zb)
        pltpu.sync_copy(zb, out_hbm.at[c, pl.ds(off, CH)])
        return 0

    lax.fori_loop(0, ntrips, z, 0)


def _sc_degree_body(dst_hbm, out_hbm, idx_d0, idx_d1, ones_v, zb, accd,
                    sem0, sem1):
    c = lax.axis_index("c")
    s = lax.axis_index("s")
    wid = s * NC + c

    _fill_rows(ones_v, K, DD, 1.0)
    _zero_rows(zb, CH, DD)

    _zero_acc(s, zb, accd)
    plsc.subcore_barrier()

    base = wid * EPW

    def start(j, idx_d, sem):
        pltpu.async_copy(dst_hbm.at[pl.ds(base + j * K, K)], idx_d, sem)

    def finish(j, idx_d, sem):
        pltpu.make_async_copy(dst_hbm.at[pl.ds(base + j * K, K)],
                              idx_d, sem).wait()
        pltpu.sync_copy(ones_v, accd.at[idx_d], add=True)

    start(0, idx_d0, sem0)

    def pair(i, _):
        start(2 * i + 1, idx_d1, sem1)
        finish(2 * i, idx_d0, sem0)
        start(2 * i + 2, idx_d0, sem0)
        finish(2 * i + 1, idx_d1, sem1)
        return 0

    lax.fori_loop(0, (NCHUNK - 1) // 2, pair, 0)
    finish(NCHUNK - 1, idx_d0, sem0)
    plsc.subcore_barrier()

    _writeout_acc(c, s, zb, accd, out_hbm)


_sc_degree = pl.kernel(
    _sc_degree_body,
    out_type=jax.ShapeDtypeStruct((NC, N_NODES, DD), jnp.float32),
    mesh=_mesh,
    scratch_types=[
        pltpu.VMEM((K,), jnp.int32),
        pltpu.VMEM((K,), jnp.int32),
        pltpu.VMEM((K, DD), jnp.float32),
        pltpu.VMEM((CH, DD), jnp.float32),
        pltpu.VMEM_SHARED((N_NODES, DD), jnp.float32),
        pltpu.SemaphoreType.DMA,
        pltpu.SemaphoreType.DMA,
    ],
)


NIB = 4
NPRE = NCHUNK - 1


def _sc_aggregate_body(t_hbm, src_hbm, dst_hbm, out_hbm,
                       idx_s, idx_d, rows0, rows1, zb, acc,
                       isem0, isem1, isem2, isem3, gsem0, gsem1):
    c = lax.axis_index("c")
    s = lax.axis_index("s")
    wid = s * NC + c

    _zero_rows(zb, CH, D)
    _zero_acc(s, zb, acc)
    plsc.subcore_barrier()

    base = wid * EPW
    rows = (rows0, rows1)
    gsem = (gsem0, gsem1)
    isem = (isem0, isem1, isem2, isem3)

    def start_idx(j, p):
        e = base + jnp.minimum(j, NCHUNK - 1) * K
        pltpu.async_copy(src_hbm.at[pl.ds(e, K)], idx_s.at[p], isem[p])
        pltpu.async_copy(dst_hbm.at[pl.ds(e, K)], idx_d.at[p], isem[p])

    def start_gather(p, r):
        pltpu.make_async_copy(src_hbm.at[pl.ds(base, K)],
                              idx_s.at[p], isem[p]).wait()
        pltpu.make_async_copy(dst_hbm.at[pl.ds(base, K)],
                              idx_d.at[p], isem[p]).wait()
        pltpu.async_copy(t_hbm.at[idx_s.at[p]], rows[r], gsem[r])

    def finish(p, r):
        pltpu.make_async_copy(t_hbm.at[idx_s.at[p]], rows[r], gsem[r]).wait()
        pltpu.sync_copy(rows[r], acc.at[idx_d.at[p]], add=True)

    for p in range(NIB):
        start_idx(p, p)
    start_gather(0, 0)

    def quad(i, _):
        for dj in range(NIB):
            j = NIB * i + dj
            start_gather((dj + 1) % NIB, (dj + 1) % 2)
            finish(dj, dj % 2)
            start_idx(j + NIB, dj)
        return 0

    lax.fori_loop(0, NPRE // NIB, quad, 0)
    finish(0, 0)
    for p in range(1, NIB):
        pltpu.make_async_copy(src_hbm.at[pl.ds(base, K)],
                              idx_s.at[p], isem[p]).wait()
        pltpu.make_async_copy(dst_hbm.at[pl.ds(base, K)],
                              idx_d.at[p], isem[p]).wait()
    plsc.subcore_barrier()

    _writeout_acc(c, s, zb, acc, out_hbm)


_sc_aggregate = pl.kernel(
    _sc_aggregate_body,
    out_type=jax.ShapeDtypeStruct((NC, N_NODES, D), jnp.float32),
    mesh=_mesh,
    scratch_types=[
        pltpu.VMEM((NIB, K), jnp.int32),
        pltpu.VMEM((NIB, K), jnp.int32),
        pltpu.VMEM((K, D), jnp.float32),
        pltpu.VMEM((K, D), jnp.float32),
        pltpu.VMEM((CH, D), jnp.float32),
        pltpu.VMEM_SHARED((N_NODES, D), jnp.float32),
        pltpu.SemaphoreType.DMA,
        pltpu.SemaphoreType.DMA,
        pltpu.SemaphoreType.DMA,
        pltpu.SemaphoreType.DMA,
        pltpu.SemaphoreType.DMA,
        pltpu.SemaphoreType.DMA,
    ],
)


BR = 1000
GRID = N_NODES // BR


def _dis_block(degp_ref):
    d = 1.0 + degp_ref[0, :, 0:1] + degp_ref[1, :, 0:1]
    return lax.rsqrt(d)


def _t1_body(x_ref, w_ref, degp_ref, o_ref):
    dis = _dis_block(degp_ref)
    o_ref[...] = dis * jnp.dot(x_ref[...], w_ref[...],
                               preferred_element_type=jnp.float32)


def _t2_body(p_ref, t_ref, degp_ref, b_ref, w_ref, o_ref):
    dis = _dis_block(degp_ref)
    h = p_ref[0] + p_ref[1] + t_ref[...]
    h = jnp.maximum(dis * h + b_ref[...], 0.0)
    o_ref[...] = dis * jnp.dot(h, w_ref[...],
                               preferred_element_type=jnp.float32)


def _t3_body(p_ref, t_ref, degp_ref, b_ref, w_ref, bp_ref, o_ref):
    dis = _dis_block(degp_ref)
    h = p_ref[0] + p_ref[1] + t_ref[...]
    h = jnp.maximum(dis * h + b_ref[...], 0.0)
    o_ref[...] = jnp.dot(h, w_ref[...],
                         preferred_element_type=jnp.float32) + bp_ref[...]


def _spec_rows(width):
    return pl.BlockSpec((BR, width), lambda i: (i, 0))


def _spec_pair(width):
    return pl.BlockSpec((NC, BR, width), lambda i: (0, i, 0))


def _spec_full(shape):
    return pl.BlockSpec(shape, lambda i: tuple(0 for _ in shape))


_t1 = pl.pallas_call(
    _t1_body,
    grid=(GRID,),
    in_specs=[_spec_rows(D), _spec_full((D, D)), _spec_pair(DD)],
    out_specs=_spec_rows(D),
    out_shape=jax.ShapeDtypeStruct((N_NODES, D), jnp.float32),
)

_t2 = pl.pallas_call(
    _t2_body,
    grid=(GRID,),
    in_specs=[_spec_pair(D), _spec_rows(D), _spec_pair(DD),
              _spec_full((1, D)), _spec_full((D, D))],
    out_specs=_spec_rows(D),
    out_shape=jax.ShapeDtypeStruct((N_NODES, D), jnp.float32),
)

_t3 = pl.pallas_call(
    _t3_body,
    grid=(GRID,),
    in_specs=[_spec_pair(D), _spec_rows(D), _spec_pair(DD),
              _spec_full((1, D)), _spec_full((D, 1)), _spec_full((1, 1))],
    out_specs=_spec_rows(1),
    out_shape=jax.ShapeDtypeStruct((N_NODES, 1), jnp.float32),
)


@jax.jit
def kernel(x, edge_index, W1, b1, W2, b2, Wp, bp):
    src = edge_index[0].astype(jnp.int32)
    dst = edge_index[1].astype(jnp.int32)
    b1r = b1.reshape(1, D)
    b2r = b2.reshape(1, D)
    bpr = bp.reshape(1, 1)

    degp = _sc_degree(dst)
    t1 = _t1(x, W1, degp)
    p1 = _sc_aggregate(t1, src, dst)
    t2 = _t2(p1, t1, degp, b1r, W2)
    p2 = _sc_aggregate(t2, src, dst)
    out = _t3(p2, t2, degp, b2r, Wp, bpr)
    return out


# speedup vs baseline: 25.2250x; 1.0019x over previous
import functools

import jax
import jax.numpy as jnp
from jax import lax
from jax.experimental import pallas as pl
from jax.experimental.pallas import tpu as pltpu
from jax.experimental.pallas import tpu_sc as plsc

N_NODES = 10000
N_EDGES = 320000
D = 128
DD = 128

NC = 2
NS = 16
NW = NC * NS
EPW = N_EDGES // NW
K = 80
NCHUNK = EPW // K
RPT = 624
CH = 16

_mesh = plsc.VectorSubcoreMesh(core_axis_name="c", subcore_axis_name="s")


def _fill_rows(ref, nrows, width, value):
    v = jnp.full((16,), value, jnp.float32)
    groups = width // 16

    def body(i, _):
        r = i // groups
        g = i % groups
        ref[r, pl.ds(g * 16, 16)] = v
        return 0

    lax.fori_loop(0, nrows * groups, body, 0)


def _zero_rows(ref, nrows, width):
    _fill_rows(ref, nrows, width, 0.0)


def _tile_rows(s):
    base = s * RPT
    ntrips = jnp.where(s == NS - 1, (N_NODES - (NS - 1) * RPT) // CH,
                       RPT // CH)
    return base, ntrips


def _zero_acc(s, zb, acc):
    base, ntrips = _tile_rows(s)

    def z(i, _):
        pltpu.sync_copy(zb, acc.at[pl.ds(base + i * CH, CH)])
        return 0

    lax.fori_loop(0, ntrips, z, 0)


def _writeout_acc(c, s, zb, acc, out_hbm):
    base, ntrips = _tile_rows(s)

    def z(i, _):
        off = base + i * CH
        pltpu.sync_copy(acc.at[pl.ds(off, CH)], zb)
        pltpu.sync_copy(zb, out_hbm.at[c, pl.ds(off, CH)])
        return 0

    lax.fori_loop(0, ntrips, z, 0)


def _sc_degree_body(dst_hbm, out_hbm, idx_d0, idx_d1, ones_v, zb, accd,
                    sem0, sem1):
    c = lax.axis_index("c")
    s = lax.axis_index("s")
    wid = s * NC + c

    _fill_rows(ones_v, K, DD, 1.0)
    _zero_rows(zb, CH, DD)

    _zero_acc(s, zb, accd)
    plsc.subcore_barrier()

    base = wid * EPW

    def start(j, idx_d, sem):
        pltpu.async_copy(dst_hbm.at[pl.ds(base + j * K, K)], idx_d, sem)

    def finish(j, idx_d, sem):
        pltpu.make_async_copy(dst_hbm.at[pl.ds(base + j * K, K)],
                              idx_d, sem).wait()
        pltpu.sync_copy(ones_v, accd.at[idx_d], add=True)

    start(0, idx_d0, sem0)

    def pair(i, _):
        start(2 * i + 1, idx_d1, sem1)
        finish(2 * i, idx_d0, sem0)
        start(2 * i + 2, idx_d0, sem0)
        finish(2 * i + 1, idx_d1, sem1)
        return 0

    lax.fori_loop(0, (NCHUNK - 1) // 2, pair, 0)
    finish(NCHUNK - 1, idx_d0, sem0)
    plsc.subcore_barrier()

    _writeout_acc(c, s, zb, accd, out_hbm)


_sc_degree = pl.kernel(
    _sc_degree_body,
    out_type=jax.ShapeDtypeStruct((NC, N_NODES, DD), jnp.float32),
    mesh=_mesh,
    scratch_types=[
        pltpu.VMEM((K,), jnp.int32),
        pltpu.VMEM((K,), jnp.int32),
        pltpu.VMEM((K, DD), jnp.float32),
        pltpu.VMEM((CH, DD), jnp.float32),
        pltpu.VMEM_SHARED((N_NODES, DD), jnp.float32),
        pltpu.SemaphoreType.DMA,
        pltpu.SemaphoreType.DMA,
    ],
)


NIB = 4
NPRE = NCHUNK - 1


def _sc_aggregate_body(t_hbm, src_hbm, dst_hbm, out_hbm,
                       idx_s, idx_d, rows0, rows1, zb, acc,
                       isem0, isem1, isem2, isem3,
                       gsem0, gsem1, ssem0, ssem1):
    c = lax.axis_index("c")
    s = lax.axis_index("s")
    wid = s * NC + c

    _zero_rows(zb, CH, D)
    _zero_acc(s, zb, acc)
    plsc.subcore_barrier()

    base = wid * EPW
    rows = (rows0, rows1)
    gsem = (gsem0, gsem1)
    ssem = (ssem0, ssem1)
    isem = (isem0, isem1, isem2, isem3)

    def sidx(j, p):
        e = base + j * K
        pltpu.async_copy(src_hbm.at[pl.ds(e, K)], idx_s.at[p], isem[p])
        pltpu.async_copy(dst_hbm.at[pl.ds(e, K)], idx_d.at[p], isem[p])

    def sgather(p, r):
        pltpu.make_async_copy(src_hbm.at[pl.ds(base, K)],
                              idx_s.at[p], isem[p]).wait()
        pltpu.make_async_copy(dst_hbm.at[pl.ds(base, K)],
                              idx_d.at[p], isem[p]).wait()
        pltpu.async_copy(t_hbm.at[idx_s.at[p]], rows[r], gsem[r])

    def sscatter(p, r):
        pltpu.make_async_copy(t_hbm.at[idx_s.at[p]], rows[r], gsem[r]).wait()
        pltpu.async_copy(rows[r], acc.at[idx_d.at[p]], ssem[r], add=True)

    def wscatter(p, r):
        pltpu.make_async_copy(rows[r], acc.at[idx_d.at[p]], ssem[r]).wait()

    for p in range(NIB):
        sidx(p, p)
    sgather(0, 0)
    sgather(1, 1)
    sscatter(0, 0)

    def quad(i, _):
        for dj in range(1, 5):
            j = NIB * i + dj
            wscatter((dj - 1) % 4, (dj - 1) % 2)
            sgather((dj + 1) % 4, (dj + 1) % 2)
            sscatter(dj % 4, dj % 2)
            sidx(j + 3, (dj + 3) % 4)
        return 0

    lax.fori_loop(0, 30, quad, 0)
    wscatter(0, 0); sgather(2, 0); sscatter(1, 1); sidx(NCHUNK - 1, 0)
    wscatter(1, 1); sgather(3, 1); sscatter(2, 0)
    wscatter(2, 0); sgather(0, 0); sscatter(3, 1)
    wscatter(3, 1); sscatter(0, 0)
    wscatter(0, 0)
    plsc.subcore_barrier()

    _writeout_acc(c, s, zb, acc, out_hbm)


_sc_aggregate = pl.kernel(
    _sc_aggregate_body,
    out_type=jax.ShapeDtypeStruct((NC, N_NODES, D), jnp.float32),
    mesh=_mesh,
    scratch_types=[
        pltpu.VMEM((NIB, K), jnp.int32),
        pltpu.VMEM((NIB, K), jnp.int32),
        pltpu.VMEM((K, D), jnp.float32),
        pltpu.VMEM((K, D), jnp.float32),
        pltpu.VMEM((CH, D), jnp.float32),
        pltpu.VMEM_SHARED((N_NODES, D), jnp.float32),
        pltpu.SemaphoreType.DMA,
        pltpu.SemaphoreType.DMA,
        pltpu.SemaphoreType.DMA,
        pltpu.SemaphoreType.DMA,
        pltpu.SemaphoreType.DMA,
        pltpu.SemaphoreType.DMA,
        pltpu.SemaphoreType.DMA,
        pltpu.SemaphoreType.DMA,
    ],
)


BR = 1000
GRID = N_NODES // BR


def _dis_block(degp_ref):
    d = 1.0 + degp_ref[0, :, 0:1] + degp_ref[1, :, 0:1]
    return lax.rsqrt(d)


def _t1_body(x_ref, w_ref, degp_ref, o_ref):
    dis = _dis_block(degp_ref)
    o_ref[...] = dis * jnp.dot(x_ref[...], w_ref[...],
                               preferred_element_type=jnp.float32)


def _t2_body(p_ref, t_ref, degp_ref, b_ref, w_ref, o_ref):
    dis = _dis_block(degp_ref)
    h = p_ref[0] + p_ref[1] + t_ref[...]
    h = jnp.maximum(dis * h + b_ref[...], 0.0)
    o_ref[...] = dis * jnp.dot(h, w_ref[...],
                               preferred_element_type=jnp.float32)


def _t3_body(p_ref, t_ref, degp_ref, b_ref, w_ref, bp_ref, o_ref):
    dis = _dis_block(degp_ref)
    h = p_ref[0] + p_ref[1] + t_ref[...]
    h = jnp.maximum(dis * h + b_ref[...], 0.0)
    o_ref[...] = jnp.dot(h, w_ref[...],
                         preferred_element_type=jnp.float32) + bp_ref[...]


def _spec_rows(width):
    return pl.BlockSpec((BR, width), lambda i: (i, 0))


def _spec_pair(width):
    return pl.BlockSpec((NC, BR, width), lambda i: (0, i, 0))


def _spec_full(shape):
    return pl.BlockSpec(shape, lambda i: tuple(0 for _ in shape))


_t1 = pl.pallas_call(
    _t1_body,
    grid=(GRID,),
    in_specs=[_spec_rows(D), _spec_full((D, D)), _spec_pair(DD)],
    out_specs=_spec_rows(D),
    out_shape=jax.ShapeDtypeStruct((N_NODES, D), jnp.float32),
)

_t2 = pl.pallas_call(
    _t2_body,
    grid=(GRID,),
    in_specs=[_spec_pair(D), _spec_rows(D), _spec_pair(DD),
              _spec_full((1, D)), _spec_full((D, D))],
    out_specs=_spec_rows(D),
    out_shape=jax.ShapeDtypeStruct((N_NODES, D), jnp.float32),
)

_t3 = pl.pallas_call(
    _t3_body,
    grid=(GRID,),
    in_specs=[_spec_pair(D), _spec_rows(D), _spec_pair(DD),
              _spec_full((1, D)), _spec_full((D, 1)), _spec_full((1, 1))],
    out_specs=_spec_rows(1),
    out_shape=jax.ShapeDtypeStruct((N_NODES, 1), jnp.float32),
)


@jax.jit
def kernel(x, edge_index, W1, b1, W2, b2, Wp, bp):
    src = edge_index[0].astype(jnp.int32)
    dst = edge_index[1].astype(jnp.int32)
    b1r = b1.reshape(1, D)
    b2r = b2.reshape(1, D)
    bpr = bp.reshape(1, 1)

    degp = _sc_degree(dst)
    t1 = _t1(x, W1, degp)
    p1 = _sc_aggregate(t1, src, dst)
    t2 = _t2(p1, t1, degp, b1r, W2)
    p2 = _sc_aggregate(t2, src, dst)
    out = _t3(p2, t2, degp, b2r, Wp, bpr)
    return out
